# baseline (speedup 1.0000x reference)
import functools

import jax
import jax.numpy as jnp
from jax import lax
from jax.experimental import pallas as pl
from jax.experimental.pallas import tpu as pltpu
from jax.experimental.pallas import tpu_sc as plsc

def _permute(x, perm):
    return lax.gather(
        x, perm[:, None],
        dimension_numbers=lax.GatherDimensionNumbers(
            offset_dims=(), collapsed_slice_dims=(0,), start_index_map=(0,)),
        slice_sizes=(1,),
        mode=lax.GatherScatterMode.PROMISE_IN_BOUNDS)


_NC = 2
_NS = 16
_W = _NC * _NS
_LANES = 16
_NCHUNK = 2
_UNROLL = 5


def _build(S, N, D, interpret=False):
    RPW = S // _W
    C = N // _NCHUNK
    VR = C // _LANES
    ITERS = VR // _UNROLL
    assert S % _W == 0 and N % (_NCHUNK * _LANES * _UNROLL) == 0

    mesh = plsc.VectorSubcoreMesh(
        core_axis_name="c", subcore_axis_name="s",
        num_cores=_NC, num_subcores=_NS)

    @functools.partial(
        pl.kernel,
        out_type=(jax.ShapeDtypeStruct((S, D), jnp.float32),
                  jax.ShapeDtypeStruct((S,), jnp.float32)),
        mesh=mesh,
        scratch_types=[
            pltpu.VMEM((C,), jnp.float32),
            pltpu.VMEM((C,), jnp.float32),
            pltpu.VMEM((RPW,), jnp.int32),
            pltpu.VMEM((RPW,), jnp.float32),
            pltpu.VMEM((RPW, D), jnp.float32),
            pltpu.SemaphoreType.DMA,
            pltpu.SemaphoreType.DMA,
            pltpu.SemaphoreType.DMA,
        ],
        compiler_params=pltpu.CompilerParams(use_tc_tiling_on_sc=False),
        interpret=interpret,
    )
    def run(obj_hbm, x_hbm, outx_hbm, outs_hbm,
            buf0, buf1, idxv, valv, xrows, sem0, sem1, gsem):
        wid = lax.axis_index("s") * _NC + lax.axis_index("c")
        row0 = wid * RPW
        bufs = (buf0, buf1)
        sems = (sem0, sem1)
        lane = lax.iota(jnp.int32, _LANES)
        ntasks = RPW * _NCHUNK

        cps = {}

        def start(t):
            r, cc = divmod(t, _NCHUNK)
            cps[t] = pltpu.async_copy(
                obj_hbm.at[pl.ds((row0 + r) * N + cc * C, C)],
                bufs[t % 2], sems[t % 2])

        start(0)
        idxvec = jnp.zeros((_LANES,), jnp.int32)
        valvec = jnp.zeros((_LANES,), jnp.float32)

        for r in range(RPW):
            m = jnp.full((_LANES,), -jnp.inf, jnp.float32)
            mi = jnp.zeros((_LANES,), jnp.int32)
            for cc in range(_NCHUNK):
                t = r * _NCHUNK + cc
                if t + 1 < ntasks:
                    start(t + 1)
                cps[t].wait()
                buf = bufs[t % 2]
                coff = cc * VR

                def body(j, carry, buf=buf, coff=coff):
                    m, mi = carry
                    for u in range(_UNROLL):
                        k = j * _UNROLL + u
                        v = buf[pl.ds(k * _LANES, _LANES)]
                        cmp = v > m
                        m = jnp.where(cmp, v, m)
                        kv = jnp.broadcast_to(
                            (coff + k).astype(jnp.int32), (_LANES,))
                        mi = jnp.where(cmp, kv, mi)
                    return m, mi

                m, mi = lax.fori_loop(
                    0, ITERS, body, (m, mi))

            g = mi * _LANES + lane
            for sh in (8, 4, 2, 1):
                perm = jnp.bitwise_xor(lane, jnp.int32(sh))
                m2 = _permute(m, perm)
                g2 = _permute(g, perm)
                take2 = (m2 > m) | ((m2 == m) & (g2 < g))
                m = jnp.where(take2, m2, m)
                g = jnp.where(take2, g2, g)
            idxvec = jnp.where(lane == r, g, idxvec)
            valvec = jnp.where(lane == r, m, valvec)

        idxv[...] = idxvec
        valv[...] = valvec
        pltpu.async_copy(x_hbm.at[idxv], xrows, gsem).wait()
        pltpu.sync_copy(xrows, outx_hbm.at[pl.ds(row0, RPW)])
        pltpu.sync_copy(valv, outs_hbm.at[pl.ds(row0, RPW)])

    return run


def kernel(X, samples, num_samples):
    S, N = samples.shape[0], samples.shape[1]
    D = X.shape[-1]
    obj = samples.reshape(S * N)
    run = _build(S, N, D)
    x_samp, score = run(obj, X)
    return x_samp, score.reshape(S, 1)


# speedup vs baseline: 4.1492x; 4.1492x over previous
import functools

import jax
import jax.numpy as jnp
from jax import lax
from jax.experimental import pallas as pl
from jax.experimental.pallas import tpu as pltpu
from jax.experimental.pallas import tpu_sc as plsc

_NC = 2
_NS = 16
_W = _NC * _NS
_LANES = 16
_CHROWS = 125
_NEG = float("-inf")


def _build_partial(S, N, interpret=False):
    NPW = N // _W
    NCH = NPW // _CHROWS
    NV = S // _LANES
    QUAD = NV // 4
    CH = _CHROWS * S
    assert N % (_W * _CHROWS) == 0 and S % (4 * _LANES) == 0
    assert NCH % 2 == 1

    mesh = plsc.VectorSubcoreMesh(
        core_axis_name="c", subcore_axis_name="s",
        num_cores=_NC, num_subcores=_NS)

    @functools.partial(
        pl.kernel,
        out_type=(jax.ShapeDtypeStruct((_W * S,), jnp.float32),
                  jax.ShapeDtypeStruct((_W * S,), jnp.int32)),
        mesh=mesh,
        scratch_types=[
            pltpu.VMEM((CH,), jnp.float32),
            pltpu.VMEM((CH,), jnp.float32),
            pltpu.VMEM((S,), jnp.float32),
            pltpu.VMEM((S,), jnp.int32),
            pltpu.SemaphoreType.DMA,
            pltpu.SemaphoreType.DMA,
        ],
        compiler_params=pltpu.CompilerParams(use_tc_tiling_on_sc=False),
        interpret=interpret,
    )
    def run(obj_hbm, pmax_hbm, pidx_hbm, buf0, buf1, pmax, pidx, sem0, sem1):
        w = lax.axis_index("s") * _NC + lax.axis_index("c")
        n0 = w * NPW
        bufs = (buf0, buf1)
        sems = (sem0, sem1)

        def copy_chunk(ch, par):
            return pltpu.make_async_copy(
                obj_hbm.at[pl.ds((n0 + ch * _CHROWS) * S, CH)],
                bufs[par], sems[par])

        def consume(buf, ch):
            for q in range(4):
                ms = [pmax[pl.ds((q * QUAD + u) * _LANES, _LANES)]
                      for u in range(QUAD)]
                mis = [pidx[pl.ds((q * QUAD + u) * _LANES, _LANES)]
                       for u in range(QUAD)]

                def body(j, carry, q=q, buf=buf):
                    m8, mi8 = carry
                    m8, mi8 = list(m8), list(mi8)
                    nsplat = jnp.broadcast_to(
                        (n0 + ch * _CHROWS + j).astype(jnp.int32), (_LANES,))
                    for u in range(QUAD):
                        v = buf[pl.ds(j * S + (q * QUAD + u) * _LANES,
                                      _LANES)]
                        cmp = v > m8[u]
                        m8[u] = jnp.where(cmp, v, m8[u])
                        mi8[u] = jnp.where(cmp, nsplat, mi8[u])
                    return tuple(m8), tuple(mi8)

                ms, mis = lax.fori_loop(
                    0, _CHROWS, body, (tuple(ms), tuple(mis)))
                for u in range(QUAD):
                    pmax[pl.ds((q * QUAD + u) * _LANES, _LANES)] = ms[u]
                    pidx[pl.ds((q * QUAD + u) * _LANES, _LANES)] = mis[u]

        neg = jnp.full((_LANES,), _NEG, jnp.float32)
        zero = jnp.zeros((_LANES,), jnp.int32)
        for u in range(NV):
            pmax[pl.ds(u * _LANES, _LANES)] = neg
            pidx[pl.ds(u * _LANES, _LANES)] = zero

        copy_chunk(0, 0).start()

        def pair(p, carry):
            ch0 = 2 * p
            copy_chunk(ch0 + 1, 1).start()
            copy_chunk(ch0, 0).wait()
            consume(buf0, ch0)
            copy_chunk(ch0 + 2, 0).start()
            copy_chunk(ch0 + 1, 1).wait()
            consume(buf1, ch0 + 1)
            return carry

        lax.fori_loop(0, (NCH - 1) // 2, pair, 0)
        copy_chunk(NCH - 1, 0).wait()
        consume(buf0, NCH - 1)

        pltpu.sync_copy(pmax, pmax_hbm.at[pl.ds(w * S, S)])
        pltpu.sync_copy(pidx, pidx_hbm.at[pl.ds(w * S, S)])

    return run


def _build_merge(S, N, D, interpret=False):
    NPW = N // _W

    mesh = plsc.VectorSubcoreMesh(
        core_axis_name="c", subcore_axis_name="s",
        num_cores=_NC, num_subcores=_NS)

    @functools.partial(
        pl.kernel,
        out_type=(jax.ShapeDtypeStruct((S, D), jnp.float32),
                  jax.ShapeDtypeStruct((S,), jnp.float32)),
        mesh=mesh,
        scratch_types=[
            pltpu.VMEM((_W, _LANES), jnp.float32),
            pltpu.VMEM((_W, _LANES), jnp.int32),
            pltpu.VMEM((_LANES,), jnp.int32),
            pltpu.VMEM((_LANES,), jnp.float32),
            pltpu.VMEM((_LANES, D), jnp.float32),
            pltpu.SemaphoreType.DMA,
            pltpu.SemaphoreType.DMA,
            pltpu.SemaphoreType.DMA,
        ],
        compiler_params=pltpu.CompilerParams(use_tc_tiling_on_sc=False),
        interpret=interpret,
    )
    def run(pmax_hbm, pidx_hbm, x_hbm, outx_hbm, outs_hbm,
            mgm, mgi, idxv, valv, xrows, sem0, sem1, gsem):
        w = lax.axis_index("s") * _NC + lax.axis_index("c")
        s0 = w * _LANES

        cps = []
        for k in range(_W):
            cm = pltpu.make_async_copy(
                pmax_hbm.at[pl.ds(k * S + s0, _LANES)], mgm.at[k], sem0)
            ci = pltpu.make_async_copy(
                pidx_hbm.at[pl.ds(k * S + s0, _LANES)], mgi.at[k], sem1)
            cm.start()
            ci.start()
            cps.append((cm, ci))
        for cm, ci in cps:
            cm.wait()
            ci.wait()

        m = jnp.full((_LANES,), _NEG, jnp.float32)
        mi = jnp.zeros((_LANES,), jnp.int32)
        for k in range(_W):
            v = mgm[k]
            vi = mgi[k]
            cmp = v > m
            m = jnp.where(cmp, v, m)
            mi = jnp.where(cmp, vi, mi)

        idxv[...] = mi
        valv[...] = m
        pltpu.async_copy(x_hbm.at[idxv], xrows, gsem).wait()
        pltpu.sync_copy(xrows, outx_hbm.at[pl.ds(s0, _LANES)])
        pltpu.sync_copy(valv, outs_hbm.at[pl.ds(s0, _LANES)])

    return run


def kernel(X, samples, num_samples):
    S, N = samples.shape[0], samples.shape[1]
    D = X.shape[-1]
    obj_flat = jnp.transpose(samples, (2, 1, 0)).reshape(-1)
    pmax, pidx = _build_partial(S, N)(obj_flat)
    x_samp, score = _build_merge(S, N, D)(pmax, pidx, X)
    return x_samp, score.reshape(S, 1)
